# baseline (speedup 1.0000x reference)
import functools

import jax
import jax.numpy as jnp
from jax import lax
from jax.experimental import pallas as pl
from jax.experimental.pallas import tpu as pltpu
from jax.experimental.pallas import tpu_sc as plsc

N, T, S, D = 8, 64, 196, 384
ROWS = N * T
NUM_WORKERS = 32
ROWS_PER_W = ROWS // NUM_WORKERS
LANES = 16
DV = D // LANES


def _sc_body(x_hbm, temb_hbm, y_hbm, xbuf, tbuf):
    cid = lax.axis_index("c")
    sid = lax.axis_index("s")
    w = sid * 2 + cid
    row0 = w * ROWS_PER_W
    t0 = lax.rem(row0, T)

    pltpu.sync_copy(temb_hbm.at[pl.ds(t0, ROWS_PER_W)], tbuf)

    def per_row(i, carry):
        r = row0 + i
        pltpu.sync_copy(x_hbm.at[r], xbuf)

        tv = [tbuf[i, pl.ds(dv * LANES, LANES)] for dv in range(DV)]

        def per_s(s, c2):
            for dv in range(DV):
                sl = pl.ds(dv * LANES, LANES)
                xbuf[s, sl] = xbuf[s, sl] + tv[dv]
            return c2

        lax.fori_loop(0, S, per_s, 0, unroll=False)
        pltpu.sync_copy(xbuf, y_hbm.at[r])
        return carry

    lax.fori_loop(0, ROWS_PER_W, per_row, 0, unroll=False)


@functools.partial(jax.jit, static_argnames=())
def _sc_add(xf, T_embed):
    mesh = plsc.VectorSubcoreMesh(core_axis_name="c", subcore_axis_name="s")
    fn = pl.kernel(
        _sc_body,
        mesh=mesh,
        out_type=jax.ShapeDtypeStruct((ROWS, S, D), jnp.float32),
        scratch_types=[
            pltpu.VMEM((S, D), jnp.float32),
            pltpu.VMEM((ROWS_PER_W, D), jnp.float32),
        ],
    )
    return fn(xf, T_embed)


def kernel(x, T_embed):
    n, t_len, s, d = x.shape
    xf = x.reshape(n * t_len, s, d)
    yf = _sc_add(xf, T_embed)
    return yf.reshape(n, t_len, s, d)


# speedup vs baseline: 1.7231x; 1.7231x over previous
import functools

import jax
import jax.numpy as jnp
from jax import lax
from jax.experimental import pallas as pl
from jax.experimental.pallas import tpu as pltpu
from jax.experimental.pallas import tpu_sc as plsc

N, T, S, D = 8, 64, 196, 384
SLABS = N * S
NUM_WORKERS = 32
PER_W = SLABS // NUM_WORKERS
LANES = 16
DV = D // LANES


def _sc_body(xt_hbm, temb_hbm, y_hbm, xbuf, tbuf):
    cid = lax.axis_index("c")
    sid = lax.axis_index("s")
    w = sid * 2 + cid
    base = w * PER_W

    pltpu.sync_copy(temb_hbm, tbuf)

    def per_slab(i, carry):
        r = base + i
        pltpu.sync_copy(xt_hbm.at[r], xbuf)

        def per_row(row, c2):
            for c in range(DV):
                sl = pl.ds(c * LANES, LANES)
                xbuf[row, sl] = xbuf[row, sl] + tbuf[row, sl]
            return c2

        lax.fori_loop(0, T, per_row, 0, unroll=False)
        pltpu.sync_copy(xbuf, y_hbm.at[r])
        return carry

    lax.fori_loop(0, PER_W, per_slab, 0, unroll=False)


@jax.jit
def _sc_add(xt, T_embed):
    mesh = plsc.VectorSubcoreMesh(core_axis_name="c", subcore_axis_name="s")
    fn = pl.kernel(
        _sc_body,
        mesh=mesh,
        compiler_params=pltpu.CompilerParams(use_tc_tiling_on_sc=True),
        out_type=jax.ShapeDtypeStruct((SLABS, T, D), jnp.float32),
        scratch_types=[
            pltpu.VMEM((T, D), jnp.float32),
            pltpu.VMEM((T, D), jnp.float32),
        ],
    )
    return fn(xt, T_embed)


def kernel(x, T_embed):
    n, t_len, s, d = x.shape
    xt = jnp.transpose(x, (0, 2, 1, 3)).reshape(n * s, t_len, d)
    yt = _sc_add(xt, T_embed)
    return jnp.transpose(yt.reshape(n, s, t_len, d), (0, 2, 1, 3))


# speedup vs baseline: 2.7455x; 1.5934x over previous
import jax
import jax.numpy as jnp
from jax import lax
from jax.experimental import pallas as pl
from jax.experimental.pallas import tpu as pltpu
from jax.experimental.pallas import tpu_sc as plsc

N, T, S, D = 8, 64, 196, 384
SLABS = N * S
NUM_WORKERS = 32
PER_W = SLABS // NUM_WORKERS
LANES = 16
DV = D // LANES
ROWS_PER_STEP = 4


def _sc_body(xt_hbm, temb_hbm, y_hbm, tbuf, x0, x1, o0, o1,
             in0, in1, out0, out1):
    cid = lax.axis_index("c")
    sid = lax.axis_index("s")
    w = sid * 2 + cid
    base = w * PER_W

    pltpu.sync_copy(temb_hbm, tbuf)

    xbufs = (x0, x1)
    obufs = (o0, o1)
    in_sems = (in0, in1)
    out_sems = (out0, out1)

    def start_in(p, slab):
        pltpu.async_copy(xt_hbm.at[slab], xbufs[p], in_sems[p])

    def wait_in(p):
        pltpu.make_async_copy(xt_hbm.at[0], xbufs[p], in_sems[p]).wait()

    def start_out(p, slab):
        pltpu.async_copy(obufs[p], y_hbm.at[slab], out_sems[p])

    def wait_out(p):
        pltpu.make_async_copy(obufs[p], y_hbm.at[0], out_sems[p]).wait()

    start_in(0, base)
    start_in(1, base + 1)

    def stage(p, idx):
        xb, ob = xbufs[p], obufs[p]
        wait_in(p)

        @pl.when(idx >= 2)
        def _():
            wait_out(p)

        def per_rows(r0, c2):
            row0 = r0 * ROWS_PER_STEP
            for rr in range(ROWS_PER_STEP):
                row = row0 + rr
                for c in range(DV):
                    sl = pl.ds(c * LANES, LANES)
                    ob[row, sl] = xb[row, sl] + tbuf[row, sl]
            return c2

        lax.fori_loop(0, T // ROWS_PER_STEP, per_rows, 0, unroll=False)

        @pl.when(idx + 2 < PER_W)
        def _():
            start_in(p, base + idx + 2)

        start_out(p, base + idx)

    def body(k, carry):
        i0 = k * 2
        stage(0, i0)

        @pl.when(i0 + 1 < PER_W)
        def _():
            stage(1, i0 + 1)

        return carry

    lax.fori_loop(0, (PER_W + 1) // 2, body, 0, unroll=False)
    wait_out(0)
    wait_out(1)


@jax.jit
def _sc_add(xt, T_embed):
    mesh = plsc.VectorSubcoreMesh(core_axis_name="c", subcore_axis_name="s")
    fn = pl.kernel(
        _sc_body,
        mesh=mesh,
        compiler_params=pltpu.CompilerParams(use_tc_tiling_on_sc=True),
        out_type=jax.ShapeDtypeStruct((SLABS, T, D), jnp.float32),
        scratch_types=[
            pltpu.VMEM((T, D), jnp.float32),
            pltpu.VMEM((T, D), jnp.float32),
            pltpu.VMEM((T, D), jnp.float32),
            pltpu.VMEM((T, D), jnp.float32),
            pltpu.VMEM((T, D), jnp.float32),
            pltpu.SemaphoreType.DMA,
            pltpu.SemaphoreType.DMA,
            pltpu.SemaphoreType.DMA,
            pltpu.SemaphoreType.DMA,
        ],
    )
    return fn(xt, T_embed)


def kernel(x, T_embed):
    n, t_len, s, d = x.shape
    xt = jnp.transpose(x, (0, 2, 1, 3)).reshape(n * s, t_len, d)
    yt = _sc_add(xt, T_embed)
    return jnp.transpose(yt.reshape(n, s, t_len, d), (0, 2, 1, 3))
